# Initial kernel scaffold; baseline (speedup 1.0000x reference)
import functools

import jax
import jax.numpy as jnp
from jax import lax
from jax.experimental import pallas as pl
from jax.experimental.pallas import tpu as pltpu
from jax.experimental.pallas import tpu_sc as plsc

N = 10000
D = 128
E = 160000
NPAD = 10240
BR = 1000
CHUNK = 1280
NCHUNK = E // CHUNK
SLOTS = 4
ROWS_PER_CHUNK = CHUNK // 128


def _dense_compute(xb, w0, b0, a0, w1, b1, a1, wh0_ref, wh1_ref, s_ref):
    wh0 = jnp.dot(xb, w0, preferred_element_type=jnp.float32) + b0
    wh1 = jnp.dot(xb, w1, preferred_element_type=jnp.float32) + b1
    wh0_ref[...] = wh0
    wh1_ref[...] = wh1
    dn = (((1,), (1,)), ((), ()))
    ss0 = lax.dot_general(wh0, a0, dn, preferred_element_type=jnp.float32)
    ss1 = lax.dot_general(wh1, a1, dn, preferred_element_type=jnp.float32)
    st = jnp.concatenate([ss0, ss1], axis=1)
    stt = jnp.transpose(st, (1, 0))
    s_ref[...] = jnp.concatenate(
        [stt, jnp.zeros((4, stt.shape[1]), jnp.float32)], axis=0)


def _tc_dense1_body(x_ref, w0_ref, b0_ref, a0_ref, w1_ref, b1_ref, a1_ref,
                    wh0_ref, wh1_ref, s_ref):
    _dense_compute(x_ref[...], w0_ref[...], b0_ref[...], a0_ref[...],
                   w1_ref[...], b1_ref[...], a1_ref[...],
                   wh0_ref, wh1_ref, s_ref)


def _tc_dense2_body(p_ref, w0_ref, b0_ref, a0_ref, w1_ref, b1_ref, a1_ref,
                    wh0_ref, wh1_ref, s_ref):
    h = p_ref[0] + p_ref[1]
    xb = jnp.where(h >= 0, h, 0.01 * h)
    _dense_compute(xb, w0_ref[...], b0_ref[...], a0_ref[...],
                   w1_ref[...], b1_ref[...], a1_ref[...],
                   wh0_ref, wh1_ref, s_ref)


def _tc_combine_body(p_ref, o_ref):
    o_ref[...] = p_ref[0] + p_ref[1]


_W_SPEC = pl.BlockSpec((D, D), lambda i: (0, 0))
_B_SPEC = pl.BlockSpec((1, D), lambda i: (0, 0))
_A_SPEC = pl.BlockSpec((2, D), lambda i: (0, 0))
_ROW_SPEC = pl.BlockSpec((BR, D), lambda i: (i, 0))
_S_SPEC = pl.BlockSpec((8, BR), lambda i: (0, i))
_PART_SPEC = pl.BlockSpec((2, BR, D), lambda i: (0, i, 0))

_DENSE_OUT = (
    jax.ShapeDtypeStruct((N, D), jnp.float32),
    jax.ShapeDtypeStruct((N, D), jnp.float32),
    jax.ShapeDtypeStruct((8, N), jnp.float32),
)

_tc_dense1 = pl.pallas_call(
    _tc_dense1_body,
    grid=(N // BR,),
    in_specs=[_ROW_SPEC, _W_SPEC, _B_SPEC, _A_SPEC, _W_SPEC, _B_SPEC, _A_SPEC],
    out_specs=[_ROW_SPEC, _ROW_SPEC, _S_SPEC],
    out_shape=_DENSE_OUT,
)

_tc_dense2 = pl.pallas_call(
    _tc_dense2_body,
    grid=(N // BR,),
    in_specs=[_PART_SPEC, _W_SPEC, _B_SPEC, _A_SPEC, _W_SPEC, _B_SPEC, _A_SPEC],
    out_specs=[_ROW_SPEC, _ROW_SPEC, _S_SPEC],
    out_shape=_DENSE_OUT,
)

_tc_combine = pl.pallas_call(
    _tc_combine_body,
    grid=(N // BR,),
    in_specs=[_PART_SPEC],
    out_specs=_ROW_SPEC,
    out_shape=jax.ShapeDtypeStruct((N, D), jnp.float32),
)


def _sc_layer_body(wh0_hbm, wh1_hbm, s_hbm, ei0_hbm, ei1_hbm, out_hbm,
                   s1b, s2b,
                   src0, dst0, p0, src1, dst1, p1,
                   rows, numer, den0, den1, sem):
    core = lax.axis_index("core")
    sub = lax.axis_index("subcore")
    wid = core * 16 + sub

    zv = jnp.zeros((16,), jnp.float32)

    @pl.loop(0, 128)
    def _(i):
        @pl.loop(0, D, step=16)
        def _(q):
            rows[i, pl.ds(q, 16)] = zv

    @pl.loop(0, 640, step=16)
    def _(q):
        p0[pl.ds(q, 16)] = zv

    nbase = sub * 640
    for k in range(5):
        pltpu.sync_copy(rows.at[:, :], numer.at[pl.ds(nbase + 128 * k, 128), :])
    pltpu.sync_copy(p0.at[pl.ds(0, 640)], den0.at[pl.ds(nbase, 640)])
    pltpu.sync_copy(p0.at[pl.ds(0, 640)], den1.at[pl.ds(nbase, 640)])
    plsc.subcore_barrier()

    for r, (ei, src, dst, p, den) in enumerate(
            ((ei0_hbm, src0, dst0, p0, den0), (ei1_hbm, src1, dst1, p1, den1))):
        pltpu.sync_copy(s_hbm.at[2 * r, pl.ds(0, N)], s1b.at[pl.ds(0, N)])
        pltpu.sync_copy(s_hbm.at[2 * r + 1, pl.ds(0, N)], s2b.at[pl.ds(0, N)])

        def _vmax(i, m):
            return jnp.maximum(m, s1b[pl.ds(16 * i, 16)])

        def _vmax2(i, m):
            return jnp.maximum(m, s2b[pl.ds(16 * i, 16)])

        m1 = lax.fori_loop(0, N // 16, _vmax, jnp.full((16,), -jnp.inf))
        m2 = lax.fori_loop(0, N // 16, _vmax2, jnp.full((16,), -jnp.inf))
        msum = jnp.max(m1) + jnp.max(m2)
        c_r = jnp.where(msum >= 0, msum, 0.01 * msum)

        for l in range(SLOTS):
            cid = wid + 32 * l
            lbase = l * CHUNK

            @pl.when(cid < NCHUNK)
            def _():
                ebase = cid * CHUNK
                pltpu.sync_copy(ei.at[0, pl.ds(ebase, CHUNK)],
                                src.at[pl.ds(lbase, CHUNK)])

                @pl.loop(0, ROWS_PER_CHUNK)
                def _(j):
                    pltpu.sync_copy(ei.at[1, pl.ds(ebase + 128 * j, 128)],
                                    dst.at[ROWS_PER_CHUNK * l + j])

                @pl.loop(0, ROWS_PER_CHUNK)
                def _(j):
                    @pl.loop(0, 128, step=16)
                    def _(t):
                        off = lbase + 128 * j + t
                        sv = src[pl.ds(off, 16)]
                        dv = dst[ROWS_PER_CHUNK * l + j, pl.ds(t, 16)]
                        e = (plsc.load_gather(s1b, [sv])
                             + plsc.load_gather(s2b, [dv]))
                        e = jnp.where(e >= 0, e, 0.01 * e)
                        p[pl.ds(off, 16)] = jnp.exp(e - c_r)

                    pltpu.sync_copy(p.at[pl.ds(lbase + 128 * j, 128)],
                                    den.at[dst.at[ROWS_PER_CHUNK * l + j]],
                                    add=True)

    plsc.subcore_barrier()

    for den in (den0, den1):
        pltpu.sync_copy(den.at[pl.ds(nbase, 640)], s1b.at[pl.ds(0, 640)])

        @pl.loop(0, 640, step=16)
        def _(q):
            s1b[pl.ds(q, 16)] = 1.0 / s1b[pl.ds(q, 16)]

        pltpu.sync_copy(s1b.at[pl.ds(0, 640)], den.at[pl.ds(nbase, 640)])
    plsc.subcore_barrier()

    pltpu.sync_copy(den0.at[pl.ds(0, NPAD)], s1b.at[pl.ds(0, NPAD)])
    pltpu.sync_copy(den1.at[pl.ds(0, NPAD)], s2b.at[pl.ds(0, NPAD)])

    for (invb, src, dst, p) in ((s1b, src0, dst0, p0), (s2b, src1, dst1, p1)):
        for l in range(SLOTS):
            cid = wid + 32 * l
            lbase = l * CHUNK

            @pl.when(cid < NCHUNK)
            def _():
                @pl.loop(0, ROWS_PER_CHUNK)
                def _(j):
                    @pl.loop(0, 128, step=16)
                    def _(t):
                        off = lbase + 128 * j + t
                        dv = dst[ROWS_PER_CHUNK * l + j, pl.ds(t, 16)]
                        iv = plsc.load_gather(invb, [dv])
                        p[pl.ds(off, 16)] = p[pl.ds(off, 16)] * iv

    for (wh, src, dst, p) in ((wh0_hbm, src0, dst0, p0),
                              (wh1_hbm, src1, dst1, p1)):
        for l in range(SLOTS):
            cid = wid + 32 * l
            lbase = l * CHUNK

            @pl.when(cid < NCHUNK)
            def _():
                @pl.loop(0, ROWS_PER_CHUNK)
                def _(j):
                    off = lbase + 128 * j
                    pltpu.async_copy(wh.at[src.at[pl.ds(off, 128)]],
                                     rows, sem).wait()

                    @pl.loop(0, 128)
                    def _(i):
                        a = p[off + i]
                        for q in range(D // 16):
                            rows[i, pl.ds(16 * q, 16)] = (
                                rows[i, pl.ds(16 * q, 16)] * a)

                    pltpu.sync_copy(rows.at[:, :],
                                    numer.at[dst.at[ROWS_PER_CHUNK * l + j]],
                                    add=True)

    plsc.subcore_barrier()

    obase = sub * 625
    pltpu.sync_copy(numer.at[pl.ds(obase, 625), :],
                    out_hbm.at[core, pl.ds(obase, 625), :])


_sc_mesh = plsc.VectorSubcoreMesh(core_axis_name="core",
                                  subcore_axis_name="subcore")

_sc_layer = pl.kernel(
    _sc_layer_body,
    out_type=jax.ShapeDtypeStruct((2, N, D), jnp.float32),
    mesh=_sc_mesh,
    scratch_types=[
        pltpu.VMEM((NPAD,), jnp.float32),
        pltpu.VMEM((NPAD,), jnp.float32),
        pltpu.VMEM((SLOTS * CHUNK,), jnp.int32),
        pltpu.VMEM((SLOTS * ROWS_PER_CHUNK, 128), jnp.int32),
        pltpu.VMEM((SLOTS * CHUNK,), jnp.float32),
        pltpu.VMEM((SLOTS * CHUNK,), jnp.int32),
        pltpu.VMEM((SLOTS * ROWS_PER_CHUNK, 128), jnp.int32),
        pltpu.VMEM((SLOTS * CHUNK,), jnp.float32),
        pltpu.VMEM((128, D), jnp.float32),
        pltpu.VMEM_SHARED((NPAD, D), jnp.float32),
        pltpu.VMEM_SHARED((NPAD,), jnp.float32),
        pltpu.VMEM_SHARED((NPAD,), jnp.float32),
        pltpu.SemaphoreType.DMA,
    ],
)


def kernel(x, edge_index_rel0, edge_index_rel1,
           W1_rel0, b1_rel0, a1_rel0, W1_rel1, b1_rel1, a1_rel1,
           W2_rel0, b2_rel0, a2_rel0, W2_rel1, b2_rel1, a2_rel1):
    ei0 = edge_index_rel0.astype(jnp.int32)
    ei1 = edge_index_rel1.astype(jnp.int32)

    def prep(b, a):
        return b.reshape(1, D), a.reshape(2, D)

    b10, a10 = prep(b1_rel0, a1_rel0)
    b11, a11 = prep(b1_rel1, a1_rel1)
    b20, a20 = prep(b2_rel0, a2_rel0)
    b21, a21 = prep(b2_rel1, a2_rel1)

    wh10, wh11, s1 = _tc_dense1(x, W1_rel0, b10, a10, W1_rel1, b11, a11)
    part1 = _sc_layer(wh10, wh11, s1, ei0, ei1)
    wh20, wh21, s2 = _tc_dense2(part1, W2_rel0, b20, a20, W2_rel1, b21, a21)
    part2 = _sc_layer(wh20, wh21, s2, ei0, ei1)
    return _tc_combine(part2)


# speedup vs baseline: 12.0964x; 12.0964x over previous
import dataclasses
import functools

import jax
import jax.numpy as jnp
from jax import lax
from jax.experimental import pallas as pl
from jax.experimental.pallas import tpu as pltpu
from jax.experimental.pallas import tpu_sc as plsc

N = 10000
D = 128
E = 160000
NPAD = 10240
BR = 1000
CHUNK = 1280
NCHUNK = E // CHUNK
SLOTS = 4
KMAX = 8
ROWS_PER_CHUNK = CHUNK // 128


def _dense_compute(xb, w0, b0, a0, w1, b1, a1, wh0_ref, wh1_ref,
                   s00_ref, s01_ref, s10_ref, s11_ref):
    wh0 = jnp.dot(xb, w0, preferred_element_type=jnp.float32) + b0
    wh1 = jnp.dot(xb, w1, preferred_element_type=jnp.float32) + b1
    wh0_ref[...] = wh0
    wh1_ref[...] = wh1
    dn = (((1,), (1,)), ((), ()))
    s00_ref[...] = lax.dot_general(wh0, a0[0:1, :], dn,
                                   preferred_element_type=jnp.float32)
    s01_ref[...] = lax.dot_general(wh0, a0[1:2, :], dn,
                                   preferred_element_type=jnp.float32)
    s10_ref[...] = lax.dot_general(wh1, a1[0:1, :], dn,
                                   preferred_element_type=jnp.float32)
    s11_ref[...] = lax.dot_general(wh1, a1[1:2, :], dn,
                                   preferred_element_type=jnp.float32)


def _tc_dense1_body(x_ref, w0_ref, b0_ref, a0_ref, w1_ref, b1_ref, a1_ref,
                    wh0_ref, wh1_ref, s00_ref, s01_ref, s10_ref, s11_ref):
    _dense_compute(x_ref[...], w0_ref[...], b0_ref[...], a0_ref[...],
                   w1_ref[...], b1_ref[...], a1_ref[...],
                   wh0_ref, wh1_ref, s00_ref, s01_ref, s10_ref, s11_ref)


def _tc_dense2_body(p_ref, w0_ref, b0_ref, a0_ref, w1_ref, b1_ref, a1_ref,
                    wh0_ref, wh1_ref, s00_ref, s01_ref, s10_ref, s11_ref):
    h = p_ref[0] + p_ref[1]
    xb = jnp.where(h >= 0, h, 0.01 * h)
    _dense_compute(xb, w0_ref[...], b0_ref[...], a0_ref[...],
                   w1_ref[...], b1_ref[...], a1_ref[...],
                   wh0_ref, wh1_ref, s00_ref, s01_ref, s10_ref, s11_ref)


def _tc_combine_body(p_ref, o_ref):
    o_ref[...] = p_ref[0] + p_ref[1]


_W_SPEC = pl.BlockSpec((D, D), lambda i: (0, 0))
_B_SPEC = pl.BlockSpec((1, D), lambda i: (0, 0))
_A_SPEC = pl.BlockSpec((2, D), lambda i: (0, 0))
_ROW_SPEC = pl.BlockSpec((BR, D), lambda i: (i, 0))
_S_SPEC = pl.BlockSpec((BR, 1), lambda i: (i, 0))
_PART_SPEC = pl.BlockSpec((2, BR, D), lambda i: (0, i, 0))

_DENSE_OUT = (
    jax.ShapeDtypeStruct((N, D), jnp.float32),
    jax.ShapeDtypeStruct((N, D), jnp.float32),
    jax.ShapeDtypeStruct((N, 1), jnp.float32),
    jax.ShapeDtypeStruct((N, 1), jnp.float32),
    jax.ShapeDtypeStruct((N, 1), jnp.float32),
    jax.ShapeDtypeStruct((N, 1), jnp.float32),
)

_tc_dense1 = pl.pallas_call(
    _tc_dense1_body,
    grid=(N // BR,),
    in_specs=[_ROW_SPEC, _W_SPEC, _B_SPEC, _A_SPEC, _W_SPEC, _B_SPEC, _A_SPEC],
    out_specs=[_ROW_SPEC, _ROW_SPEC, _S_SPEC, _S_SPEC, _S_SPEC, _S_SPEC],
    out_shape=_DENSE_OUT,
)

_tc_dense2 = pl.pallas_call(
    _tc_dense2_body,
    grid=(N // BR,),
    in_specs=[_PART_SPEC, _W_SPEC, _B_SPEC, _A_SPEC, _W_SPEC, _B_SPEC, _A_SPEC],
    out_specs=[_ROW_SPEC, _ROW_SPEC, _S_SPEC, _S_SPEC, _S_SPEC, _S_SPEC],
    out_shape=_DENSE_OUT,
)

_tc_combine = pl.pallas_call(
    _tc_combine_body,
    grid=(N // BR,),
    in_specs=[_PART_SPEC],
    out_specs=_ROW_SPEC,
    out_shape=jax.ShapeDtypeStruct((N, D), jnp.float32),
)


def _sc_layer_body(wh0_hbm, wh1_hbm, s00_hbm, s01_hbm, s10_hbm, s11_hbm,
                   ei0_hbm, ei1_hbm, out_hbm,
                   s1b, s2b, srcw, dstw, pbuf, ptmp, rows,
                   numer, den0, den1, sem):
    core = lax.axis_index("core")
    sub = lax.axis_index("subcore")

    zv = jnp.zeros((16,), jnp.float32)

    @pl.loop(0, 128)
    def _(i):
        @pl.loop(0, D, step=16)
        def _(q):
            rows[i, pl.ds(q, 16)] = zv

    @pl.loop(0, 640, step=16)
    def _(q):
        pbuf[pl.ds(q, 16)] = zv

    nbase = sub * 640
    for k in range(5):
        pltpu.sync_copy(rows.at[:, :], numer.at[pl.ds(nbase + 128 * k, 128), :])
    pltpu.sync_copy(pbuf.at[pl.ds(0, 640)], den0.at[pl.ds(nbase, 640)])
    pltpu.sync_copy(pbuf.at[pl.ds(0, 640)], den1.at[pl.ds(nbase, 640)])
    plsc.subcore_barrier()

    for (sa_hbm, sb_hbm, ei, wh, den) in (
            (s00_hbm, s01_hbm, ei0_hbm, wh0_hbm, den0),
            (s10_hbm, s11_hbm, ei1_hbm, wh1_hbm, den1)):
        pltpu.sync_copy(sa_hbm.at[pl.ds(0, N)], s1b.at[pl.ds(0, N)])
        pltpu.sync_copy(sb_hbm.at[pl.ds(0, N)], s2b.at[pl.ds(0, N)])

        def _vmax(i, m):
            return jnp.maximum(m, s1b[pl.ds(16 * i, 16)])

        def _vmax2(i, m):
            return jnp.maximum(m, s2b[pl.ds(16 * i, 16)])

        m1 = lax.fori_loop(0, N // 16, _vmax, jnp.full((16,), -jnp.inf))
        m2 = lax.fori_loop(0, N // 16, _vmax2, jnp.full((16,), -jnp.inf))
        msum = jnp.max(m1) + jnp.max(m2)
        c_r = jnp.where(msum >= 0, msum, 0.01 * msum)

        for k in range(KMAX):
            kc = 0 if k < SLOTS else 1
            lbase = (k % SLOTS) * CHUNK
            cid = sub + 16 * k

            @pl.when(cid < NCHUNK)
            def _():
                ebase = cid * CHUNK
                pltpu.sync_copy(ei.at[0, pl.ds(ebase, CHUNK)],
                                srcw.at[pl.ds(0, CHUNK)])

                @pl.loop(0, ROWS_PER_CHUNK)
                def _(j):
                    pltpu.sync_copy(ei.at[1, pl.ds(ebase + 128 * j, 128)],
                                    dstw.at[j])

                @pl.loop(0, ROWS_PER_CHUNK)
                def _(j):
                    @pl.loop(0, 128, step=16)
                    def _(t):
                        sv = srcw[pl.ds(128 * j + t, 16)]
                        dv = dstw[j, pl.ds(t, 16)]
                        e = (plsc.load_gather(s1b, [sv])
                             + plsc.load_gather(s2b, [dv]))
                        e = jnp.where(e >= 0, e, 0.01 * e)
                        pv = jnp.exp(e - c_r)
                        ptmp[pl.ds(t, 16)] = pv

                        @pl.when(core == kc)
                        def _():
                            pbuf[pl.ds(lbase + 128 * j + t, 16)] = pv

                    pltpu.sync_copy(ptmp.at[pl.ds(0, 128)],
                                    den.at[dstw.at[j]], add=True)

        plsc.subcore_barrier()

        pltpu.sync_copy(den.at[pl.ds(nbase, 640)], s1b.at[pl.ds(0, 640)])

        @pl.loop(0, 640, step=16)
        def _(q):
            s1b[pl.ds(q, 16)] = 1.0 / s1b[pl.ds(q, 16)]

        pltpu.sync_copy(s1b.at[pl.ds(0, 640)], den.at[pl.ds(nbase, 640)])
        plsc.subcore_barrier()

        pltpu.sync_copy(den.at[pl.ds(0, NPAD)], s1b.at[pl.ds(0, NPAD)])

        for k in range(KMAX):
            kc = 0 if k < SLOTS else 1
            lbase = (k % SLOTS) * CHUNK
            cid = sub + 16 * k

            @pl.when(jnp.logical_and(core == kc, cid < NCHUNK))
            def _():
                ebase = cid * CHUNK
                pltpu.sync_copy(ei.at[0, pl.ds(ebase, CHUNK)],
                                srcw.at[pl.ds(0, CHUNK)])

                @pl.loop(0, ROWS_PER_CHUNK)
                def _(j):
                    pltpu.sync_copy(ei.at[1, pl.ds(ebase + 128 * j, 128)],
                                    dstw.at[j])

                @pl.loop(0, ROWS_PER_CHUNK)
                def _(j):
                    @pl.loop(0, 128, step=16)
                    def _(t):
                        off = lbase + 128 * j + t
                        dv = dstw[j, pl.ds(t, 16)]
                        iv = plsc.load_gather(s1b, [dv])
                        pbuf[pl.ds(off, 16)] = pbuf[pl.ds(off, 16)] * iv

                @pl.loop(0, ROWS_PER_CHUNK)
                def _(j):
                    off = lbase + 128 * j
                    pltpu.async_copy(wh.at[srcw.at[pl.ds(128 * j, 128)]],
                                     rows, sem).wait()

                    @pl.loop(0, 128)
                    def _(i):
                        av = plsc.load_gather(
                            pbuf, [jnp.full((16,), off + i, jnp.int32)])
                        for q in range(D // 16):
                            rows[i, pl.ds(16 * q, 16)] = (
                                rows[i, pl.ds(16 * q, 16)] * av)

                    pltpu.sync_copy(rows.at[:, :],
                                    numer.at[dstw.at[j]], add=True)

    plsc.subcore_barrier()

    @pl.when(sub < 15)
    def _():
        ob = sub * 624
        pltpu.sync_copy(numer.at[pl.ds(ob, 624), :],
                        out_hbm.at[core, pl.ds(ob, 624), :])

    @pl.when(sub == 15)
    def _():
        pltpu.sync_copy(numer.at[pl.ds(9360, 640), :],
                        out_hbm.at[core, pl.ds(9360, 640), :])


_sc_mesh = plsc.VectorSubcoreMesh(core_axis_name="core",
                                  subcore_axis_name="subcore")

_sc_params = pltpu.CompilerParams()
if "needs_layout_passes" in pltpu.CompilerParams.__dataclass_fields__:
    _sc_params = dataclasses.replace(_sc_params, needs_layout_passes=False)

_sc_layer = pl.kernel(
    _sc_layer_body,
    out_type=jax.ShapeDtypeStruct((2, N, D), jnp.float32),
    mesh=_sc_mesh,
    compiler_params=_sc_params,
    scratch_types=[
        pltpu.VMEM((NPAD,), jnp.float32),
        pltpu.VMEM((NPAD,), jnp.float32),
        pltpu.VMEM((CHUNK,), jnp.int32),
        pltpu.VMEM((ROWS_PER_CHUNK, 128), jnp.int32),
        pltpu.VMEM((SLOTS * CHUNK,), jnp.float32),
        pltpu.VMEM((128,), jnp.float32),
        pltpu.VMEM((128, D), jnp.float32),
        pltpu.VMEM_SHARED((NPAD, D), jnp.float32),
        pltpu.VMEM_SHARED((NPAD,), jnp.float32),
        pltpu.VMEM_SHARED((NPAD,), jnp.float32),
        pltpu.SemaphoreType.DMA,
    ],
)


def kernel(x, edge_index_rel0, edge_index_rel1,
           W1_rel0, b1_rel0, a1_rel0, W1_rel1, b1_rel1, a1_rel1,
           W2_rel0, b2_rel0, a2_rel0, W2_rel1, b2_rel1, a2_rel1):
    ei0 = edge_index_rel0.astype(jnp.int32)
    ei1 = edge_index_rel1.astype(jnp.int32)

    def prep(b, a):
        return b.reshape(1, D), a.reshape(2, D)

    b10, a10 = prep(b1_rel0, a1_rel0)
    b11, a11 = prep(b1_rel1, a1_rel1)
    b20, a20 = prep(b2_rel0, a2_rel0)
    b21, a21 = prep(b2_rel1, a2_rel1)

    wh10, wh11, s00, s01, s10, s11 = _tc_dense1(
        x, W1_rel0, b10, a10, W1_rel1, b11, a11)
    part1 = _sc_layer(wh10, wh11, s00.reshape(N), s01.reshape(N),
                      s10.reshape(N), s11.reshape(N), ei0, ei1)
    wh20, wh21, t00, t01, t10, t11 = _tc_dense2(
        part1, W2_rel0, b20, a20, W2_rel1, b21, a21)
    part2 = _sc_layer(wh20, wh21, t00.reshape(N), t01.reshape(N),
                      t10.reshape(N), t11.reshape(N), ei0, ei1)
    return _tc_combine(part2)


# speedup vs baseline: 21.5044x; 1.7778x over previous
import dataclasses
import functools

import jax
import jax.numpy as jnp
from jax import lax
from jax.experimental import pallas as pl
from jax.experimental.pallas import tpu as pltpu
from jax.experimental.pallas import tpu_sc as plsc

N = 10000
D = 128
E = 160000
NPAD = 10240
BR = 1000
CHUNK = 1280
NCHUNK = E // CHUNK
SLOTS = 4
KMAX = 8
BATCH = 64
NB = CHUNK // BATCH


def _dense_compute(xb, w0, b0, a0, w1, b1, a1, wh0_ref, wh1_ref,
                   s00_ref, s01_ref, s10_ref, s11_ref):
    wh0 = jnp.dot(xb, w0, preferred_element_type=jnp.float32) + b0
    wh1 = jnp.dot(xb, w1, preferred_element_type=jnp.float32) + b1
    wh0_ref[...] = wh0
    wh1_ref[...] = wh1
    dn = (((1,), (1,)), ((), ()))
    s00_ref[...] = lax.dot_general(wh0, a0[0:1, :], dn,
                                   preferred_element_type=jnp.float32)
    s01_ref[...] = lax.dot_general(wh0, a0[1:2, :], dn,
                                   preferred_element_type=jnp.float32)
    s10_ref[...] = lax.dot_general(wh1, a1[0:1, :], dn,
                                   preferred_element_type=jnp.float32)
    s11_ref[...] = lax.dot_general(wh1, a1[1:2, :], dn,
                                   preferred_element_type=jnp.float32)


def _tc_dense1_body(x_ref, w0_ref, b0_ref, a0_ref, w1_ref, b1_ref, a1_ref,
                    wh0_ref, wh1_ref, s00_ref, s01_ref, s10_ref, s11_ref):
    _dense_compute(x_ref[...], w0_ref[...], b0_ref[...], a0_ref[...],
                   w1_ref[...], b1_ref[...], a1_ref[...],
                   wh0_ref, wh1_ref, s00_ref, s01_ref, s10_ref, s11_ref)


def _tc_dense2_body(p_ref, w0_ref, b0_ref, a0_ref, w1_ref, b1_ref, a1_ref,
                    wh0_ref, wh1_ref, s00_ref, s01_ref, s10_ref, s11_ref):
    h = p_ref[0] + p_ref[1]
    xb = jnp.where(h >= 0, h, 0.01 * h)
    _dense_compute(xb, w0_ref[...], b0_ref[...], a0_ref[...],
                   w1_ref[...], b1_ref[...], a1_ref[...],
                   wh0_ref, wh1_ref, s00_ref, s01_ref, s10_ref, s11_ref)


def _tc_combine_body(p_ref, o_ref):
    o_ref[...] = p_ref[0] + p_ref[1]


_W_SPEC = pl.BlockSpec((D, D), lambda i: (0, 0))
_B_SPEC = pl.BlockSpec((1, D), lambda i: (0, 0))
_A_SPEC = pl.BlockSpec((2, D), lambda i: (0, 0))
_ROW_SPEC = pl.BlockSpec((BR, D), lambda i: (i, 0))
_S_SPEC = pl.BlockSpec((BR, 1), lambda i: (i, 0))
_PART_SPEC = pl.BlockSpec((2, BR, D), lambda i: (0, i, 0))

_DENSE_OUT = (
    jax.ShapeDtypeStruct((N, D), jnp.float32),
    jax.ShapeDtypeStruct((N, D), jnp.float32),
    jax.ShapeDtypeStruct((N, 1), jnp.float32),
    jax.ShapeDtypeStruct((N, 1), jnp.float32),
    jax.ShapeDtypeStruct((N, 1), jnp.float32),
    jax.ShapeDtypeStruct((N, 1), jnp.float32),
)

_tc_dense1 = pl.pallas_call(
    _tc_dense1_body,
    grid=(N // BR,),
    in_specs=[_ROW_SPEC, _W_SPEC, _B_SPEC, _A_SPEC, _W_SPEC, _B_SPEC, _A_SPEC],
    out_specs=[_ROW_SPEC, _ROW_SPEC, _S_SPEC, _S_SPEC, _S_SPEC, _S_SPEC],
    out_shape=_DENSE_OUT,
)

_tc_dense2 = pl.pallas_call(
    _tc_dense2_body,
    grid=(N // BR,),
    in_specs=[_PART_SPEC, _W_SPEC, _B_SPEC, _A_SPEC, _W_SPEC, _B_SPEC, _A_SPEC],
    out_specs=[_ROW_SPEC, _ROW_SPEC, _S_SPEC, _S_SPEC, _S_SPEC, _S_SPEC],
    out_shape=_DENSE_OUT,
)

_tc_combine = pl.pallas_call(
    _tc_combine_body,
    grid=(N // BR,),
    in_specs=[_PART_SPEC],
    out_specs=_ROW_SPEC,
    out_shape=jax.ShapeDtypeStruct((N, D), jnp.float32),
)


def _sc_layer_body(wh0_hbm, wh1_hbm, s00_hbm, s01_hbm, s10_hbm, s11_hbm,
                   ei0_hbm, ei1_hbm, out_hbm,
                   s1b, s2b, srcw, dstflat, pbuf, ptmp, r_a, r_b,
                   numer, den0, den1, esem, gsem_a, gsem_b, ssem_a, ssem_b):
    core = lax.axis_index("core")
    sub = lax.axis_index("subcore")

    zv = jnp.zeros((16,), jnp.float32)

    @pl.loop(0, BATCH)
    def _(i):
        @pl.loop(0, D, step=16)
        def _(q):
            r_a[i, pl.ds(q, 16)] = zv

    @pl.loop(0, 640, step=16)
    def _(q):
        pbuf[pl.ds(q, 16)] = zv

    nbase = sub * 640
    for k in range(640 // BATCH):
        pltpu.sync_copy(r_a.at[:, :],
                        numer.at[pl.ds(nbase + BATCH * k, BATCH), :])
    pltpu.sync_copy(pbuf.at[pl.ds(0, 640)], den0.at[pl.ds(nbase, 640)])
    pltpu.sync_copy(pbuf.at[pl.ds(0, 640)], den1.at[pl.ds(nbase, 640)])
    plsc.subcore_barrier()

    for (sa_hbm, sb_hbm, ei, wh, den) in (
            (s00_hbm, s01_hbm, ei0_hbm, wh0_hbm, den0),
            (s10_hbm, s11_hbm, ei1_hbm, wh1_hbm, den1)):
        pltpu.sync_copy(sa_hbm.at[pl.ds(0, N)], s1b.at[pl.ds(0, N)])
        pltpu.sync_copy(sb_hbm.at[pl.ds(0, N)], s2b.at[pl.ds(0, N)])

        def _vmax(i, m):
            return jnp.maximum(m, s1b[pl.ds(16 * i, 16)])

        def _vmax2(i, m):
            return jnp.maximum(m, s2b[pl.ds(16 * i, 16)])

        m1 = lax.fori_loop(0, N // 16, _vmax, jnp.full((16,), -jnp.inf))
        m2 = lax.fori_loop(0, N // 16, _vmax2, jnp.full((16,), -jnp.inf))
        msum = jnp.max(m1) + jnp.max(m2)
        c_r = jnp.where(msum >= 0, msum, 0.01 * msum)

        for k in range(KMAX):
            kc = 0 if k < SLOTS else 1
            lbase = (k % SLOTS) * CHUNK
            cid = sub + 16 * k

            @pl.when(cid < NCHUNK)
            def _():
                ebase = cid * CHUNK
                pltpu.async_copy(ei.at[0, pl.ds(ebase, CHUNK)], srcw, esem)
                pltpu.async_copy(ei.at[1, pl.ds(ebase, CHUNK)], dstflat, esem)
                pltpu.make_async_copy(
                    ei.at[0, pl.ds(ebase, CHUNK)], srcw, esem).wait()
                pltpu.make_async_copy(
                    ei.at[1, pl.ds(ebase, CHUNK)], dstflat, esem).wait()

                @pl.loop(0, CHUNK, step=16)
                def _(t):
                    sv = srcw[pl.ds(t, 16)]
                    dv = dstflat[pl.ds(t, 16)]
                    e = (plsc.load_gather(s1b, [sv])
                         + plsc.load_gather(s2b, [dv]))
                    e = jnp.where(e >= 0, e, 0.01 * e)
                    ptmp[pl.ds(t, 16)] = jnp.exp(e - c_r)

                @pl.when(core == kc)
                def _():
                    @pl.loop(0, CHUNK, step=16)
                    def _(t):
                        pbuf[pl.ds(lbase + t, 16)] = ptmp[pl.ds(t, 16)]

                pltpu.sync_copy(ptmp, den.at[dstflat], add=True)

        plsc.subcore_barrier()

        pltpu.sync_copy(den.at[pl.ds(nbase, 640)], s1b.at[pl.ds(0, 640)])

        @pl.loop(0, 640, step=16)
        def _(q):
            s1b[pl.ds(q, 16)] = 1.0 / s1b[pl.ds(q, 16)]

        pltpu.sync_copy(s1b.at[pl.ds(0, 640)], den.at[pl.ds(nbase, 640)])
        plsc.subcore_barrier()

        pltpu.sync_copy(den.at[pl.ds(0, N)], s1b.at[pl.ds(0, N)])

        def _scale(buf, base):
            @pl.loop(0, BATCH, step=2)
            def _(i):
                av0 = plsc.load_gather(
                    pbuf, [jnp.full((16,), base + i, jnp.int32)])
                av1 = plsc.load_gather(
                    pbuf, [jnp.full((16,), base + i + 1, jnp.int32)])
                for q in range(D // 16):
                    buf[i, pl.ds(16 * q, 16)] = (
                        buf[i, pl.ds(16 * q, 16)] * av0)
                for q in range(D // 16):
                    buf[i + 1, pl.ds(16 * q, 16)] = (
                        buf[i + 1, pl.ds(16 * q, 16)] * av1)

        def _wait_gather(buf, sem):
            pltpu.make_async_copy(
                wh.at[srcw.at[pl.ds(0, BATCH)]], buf, sem).wait()

        def _wait_scatter(buf, sem):
            pltpu.make_async_copy(
                buf, numer.at[dstflat.at[pl.ds(0, BATCH)]], sem).wait()

        for k in range(KMAX):
            kc = 0 if k < SLOTS else 1
            lbase = (k % SLOTS) * CHUNK
            cid = sub + 16 * k

            @pl.when(jnp.logical_and(core == kc, cid < NCHUNK))
            def _():
                ebase = cid * CHUNK
                pltpu.async_copy(ei.at[0, pl.ds(ebase, CHUNK)], srcw, esem)
                pltpu.async_copy(ei.at[1, pl.ds(ebase, CHUNK)], dstflat, esem)
                pltpu.make_async_copy(
                    ei.at[0, pl.ds(ebase, CHUNK)], srcw, esem).wait()
                pltpu.make_async_copy(
                    ei.at[1, pl.ds(ebase, CHUNK)], dstflat, esem).wait()

                @pl.loop(0, CHUNK, step=16)
                def _(t):
                    dv = dstflat[pl.ds(t, 16)]
                    iv = plsc.load_gather(s1b, [dv])
                    pbuf[pl.ds(lbase + t, 16)] = (
                        pbuf[pl.ds(lbase + t, 16)] * iv)

                pltpu.async_copy(wh.at[srcw.at[pl.ds(0, BATCH)]], r_a, gsem_a)

                @pl.loop(0, NB, step=2)
                def _(j):
                    @pl.when(j > 0)
                    def _():
                        _wait_scatter(r_b, ssem_b)

                    pltpu.async_copy(
                        wh.at[srcw.at[pl.ds(BATCH * (j + 1), BATCH)]],
                        r_b, gsem_b)
                    _wait_gather(r_a, gsem_a)
                    _scale(r_a, lbase + BATCH * j)
                    pltpu.async_copy(
                        r_a, numer.at[dstflat.at[pl.ds(BATCH * j, BATCH)]],
                        ssem_a, add=True)

                    @pl.when(j + 2 < NB)
                    def _():
                        _wait_scatter(r_a, ssem_a)
                        pltpu.async_copy(
                            wh.at[srcw.at[pl.ds(BATCH * (j + 2), BATCH)]],
                            r_a, gsem_a)

                    _wait_gather(r_b, gsem_b)
                    _scale(r_b, lbase + BATCH * (j + 1))
                    pltpu.async_copy(
                        r_b,
                        numer.at[dstflat.at[pl.ds(BATCH * (j + 1), BATCH)]],
                        ssem_b, add=True)

                _wait_scatter(r_a, ssem_a)
                _wait_scatter(r_b, ssem_b)

    plsc.subcore_barrier()

    @pl.when(sub < 15)
    def _():
        ob = sub * 624
        pltpu.sync_copy(numer.at[pl.ds(ob, 624), :],
                        out_hbm.at[core, pl.ds(ob, 624), :])

    @pl.when(sub == 15)
    def _():
        pltpu.sync_copy(numer.at[pl.ds(9360, 640), :],
                        out_hbm.at[core, pl.ds(9360, 640), :])


_sc_mesh = plsc.VectorSubcoreMesh(core_axis_name="core",
                                  subcore_axis_name="subcore")

_sc_params = pltpu.CompilerParams()
if "needs_layout_passes" in pltpu.CompilerParams.__dataclass_fields__:
    _sc_params = dataclasses.replace(_sc_params, needs_layout_passes=False)

_sc_layer = pl.kernel(
    _sc_layer_body,
    out_type=jax.ShapeDtypeStruct((2, N, D), jnp.float32),
    mesh=_sc_mesh,
    compiler_params=_sc_params,
    scratch_types=[
        pltpu.VMEM((N,), jnp.float32),
        pltpu.VMEM((N,), jnp.float32),
        pltpu.VMEM((CHUNK,), jnp.int32),
        pltpu.VMEM((CHUNK,), jnp.int32),
        pltpu.VMEM((SLOTS * CHUNK,), jnp.float32),
        pltpu.VMEM((CHUNK,), jnp.float32),
        pltpu.VMEM((BATCH, D), jnp.float32),
        pltpu.VMEM((BATCH, D), jnp.float32),
        pltpu.VMEM_SHARED((NPAD, D), jnp.float32),
        pltpu.VMEM_SHARED((NPAD,), jnp.float32),
        pltpu.VMEM_SHARED((NPAD,), jnp.float32),
        pltpu.SemaphoreType.DMA,
        pltpu.SemaphoreType.DMA,
        pltpu.SemaphoreType.DMA,
        pltpu.SemaphoreType.DMA,
        pltpu.SemaphoreType.DMA,
    ],
)


def kernel(x, edge_index_rel0, edge_index_rel1,
           W1_rel0, b1_rel0, a1_rel0, W1_rel1, b1_rel1, a1_rel1,
           W2_rel0, b2_rel0, a2_rel0, W2_rel1, b2_rel1, a2_rel1):
    ei0 = edge_index_rel0.astype(jnp.int32)
    ei1 = edge_index_rel1.astype(jnp.int32)

    def prep(b, a):
        return b.reshape(1, D), a.reshape(2, D)

    b10, a10 = prep(b1_rel0, a1_rel0)
    b11, a11 = prep(b1_rel1, a1_rel1)
    b20, a20 = prep(b2_rel0, a2_rel0)
    b21, a21 = prep(b2_rel1, a2_rel1)

    wh10, wh11, s00, s01, s10, s11 = _tc_dense1(
        x, W1_rel0, b10, a10, W1_rel1, b11, a11)
    part1 = _sc_layer(wh10, wh11, s00.reshape(N), s01.reshape(N),
                      s10.reshape(N), s11.reshape(N), ei0, ei1)
    wh20, wh21, t00, t01, t10, t11 = _tc_dense2(
        part1, W2_rel0, b20, a20, W2_rel1, b21, a21)
    part2 = _sc_layer(wh20, wh21, t00.reshape(N), t01.reshape(N),
                      t10.reshape(N), t11.reshape(N), ei0, ei1)
    return _tc_combine(part2)


# speedup vs baseline: 24.8401x; 1.1551x over previous
import dataclasses
import functools

import jax
import jax.numpy as jnp
from jax import lax
from jax.experimental import pallas as pl
from jax.experimental.pallas import tpu as pltpu
from jax.experimental.pallas import tpu_sc as plsc

N = 10000
D = 128
E = 160000
NPAD = 10240
BR = 1000
CHUNK = 1280
NCHUNK = E // CHUNK
SLOTS = 4
KMAX = 8
BATCH = 64
NB = CHUNK // BATCH


def _dense_compute(xb, w0, b0, a0, w1, b1, a1, wh0_ref, wh1_ref,
                   s00_ref, s01_ref, s10_ref, s11_ref, c_ref):
    wh0 = jnp.dot(xb, w0, preferred_element_type=jnp.float32) + b0
    wh1 = jnp.dot(xb, w1, preferred_element_type=jnp.float32) + b1
    wh0_ref[...] = wh0
    wh1_ref[...] = wh1
    dn = (((1,), (1,)), ((), ()))
    v00 = lax.dot_general(wh0, a0[0:1, :], dn,
                          preferred_element_type=jnp.float32)
    v01 = lax.dot_general(wh0, a0[1:2, :], dn,
                          preferred_element_type=jnp.float32)
    v10 = lax.dot_general(wh1, a1[0:1, :], dn,
                          preferred_element_type=jnp.float32)
    v11 = lax.dot_general(wh1, a1[1:2, :], dn,
                          preferred_element_type=jnp.float32)
    s00_ref[...] = v00
    s01_ref[...] = v01
    s10_ref[...] = v10
    s11_ref[...] = v11
    c_blk = jnp.concatenate(
        [jnp.full((1, D), jnp.max(v)) for v in (v00, v01, v10, v11)]
        + [jnp.zeros((4, D), jnp.float32)], axis=0)
    i = pl.program_id(0)

    @pl.when(i == 0)
    def _():
        c_ref[...] = c_blk

    @pl.when(i > 0)
    def _():
        c_ref[...] = jnp.maximum(c_ref[...], c_blk)


def _tc_dense1_body(x_ref, w0_ref, b0_ref, a0_ref, w1_ref, b1_ref, a1_ref,
                    wh0_ref, wh1_ref, s00_ref, s01_ref, s10_ref, s11_ref,
                    c_ref):
    _dense_compute(x_ref[...], w0_ref[...], b0_ref[...], a0_ref[...],
                   w1_ref[...], b1_ref[...], a1_ref[...],
                   wh0_ref, wh1_ref, s00_ref, s01_ref, s10_ref, s11_ref,
                   c_ref)


def _tc_dense2_body(p_ref, w0_ref, b0_ref, a0_ref, w1_ref, b1_ref, a1_ref,
                    wh0_ref, wh1_ref, s00_ref, s01_ref, s10_ref, s11_ref,
                    c_ref):
    h = p_ref[0] + p_ref[1]
    xb = jnp.where(h >= 0, h, 0.01 * h)
    _dense_compute(xb, w0_ref[...], b0_ref[...], a0_ref[...],
                   w1_ref[...], b1_ref[...], a1_ref[...],
                   wh0_ref, wh1_ref, s00_ref, s01_ref, s10_ref, s11_ref,
                   c_ref)


def _tc_combine_body(p_ref, o_ref):
    o_ref[...] = p_ref[0] + p_ref[1]


_W_SPEC = pl.BlockSpec((D, D), lambda i: (0, 0))
_B_SPEC = pl.BlockSpec((1, D), lambda i: (0, 0))
_A_SPEC = pl.BlockSpec((2, D), lambda i: (0, 0))
_ROW_SPEC = pl.BlockSpec((BR, D), lambda i: (i, 0))
_S_SPEC = pl.BlockSpec((BR, 1), lambda i: (i, 0))
_PART_SPEC = pl.BlockSpec((2, BR, D), lambda i: (0, i, 0))

_C_SPEC = pl.BlockSpec((8, D), lambda i: (0, 0))

_DENSE_OUT = (
    jax.ShapeDtypeStruct((N, D), jnp.float32),
    jax.ShapeDtypeStruct((N, D), jnp.float32),
    jax.ShapeDtypeStruct((N, 1), jnp.float32),
    jax.ShapeDtypeStruct((N, 1), jnp.float32),
    jax.ShapeDtypeStruct((N, 1), jnp.float32),
    jax.ShapeDtypeStruct((N, 1), jnp.float32),
    jax.ShapeDtypeStruct((8, D), jnp.float32),
)

_tc_dense1 = pl.pallas_call(
    _tc_dense1_body,
    grid=(N // BR,),
    in_specs=[_ROW_SPEC, _W_SPEC, _B_SPEC, _A_SPEC, _W_SPEC, _B_SPEC, _A_SPEC],
    out_specs=[_ROW_SPEC, _ROW_SPEC, _S_SPEC, _S_SPEC, _S_SPEC, _S_SPEC,
               _C_SPEC],
    out_shape=_DENSE_OUT,
)

_tc_dense2 = pl.pallas_call(
    _tc_dense2_body,
    grid=(N // BR,),
    in_specs=[_PART_SPEC, _W_SPEC, _B_SPEC, _A_SPEC, _W_SPEC, _B_SPEC, _A_SPEC],
    out_specs=[_ROW_SPEC, _ROW_SPEC, _S_SPEC, _S_SPEC, _S_SPEC, _S_SPEC,
               _C_SPEC],
    out_shape=_DENSE_OUT,
)

_tc_combine = pl.pallas_call(
    _tc_combine_body,
    grid=(N // BR,),
    in_specs=[_PART_SPEC],
    out_specs=_ROW_SPEC,
    out_shape=jax.ShapeDtypeStruct((N, D), jnp.float32),
)


def _sc_layer_body(wh0_hbm, wh1_hbm, s00_hbm, s01_hbm, s10_hbm, s11_hbm,
                   c_hbm, ei0_hbm, ei1_hbm, out_hbm,
                   s1b, s2b, srcw, dstflat, pbuf, ptmp, cb1, cb2, r_a, r_b,
                   numer, den0, den1, esem, gsem_a, gsem_b, ssem_a, ssem_b):
    core = lax.axis_index("core")
    sub = lax.axis_index("subcore")

    zv = jnp.zeros((16,), jnp.float32)

    @pl.loop(0, BATCH)
    def _(i):
        @pl.loop(0, D, step=16)
        def _(q):
            r_a[i, pl.ds(q, 16)] = zv

    @pl.loop(0, 640, step=16)
    def _(q):
        pbuf[pl.ds(q, 16)] = zv

    nbase = sub * 640
    for k in range(640 // BATCH):
        pltpu.sync_copy(r_a.at[:, :],
                        numer.at[pl.ds(nbase + BATCH * k, BATCH), :])
    pltpu.sync_copy(pbuf.at[pl.ds(0, 640)], den0.at[pl.ds(nbase, 640)])
    pltpu.sync_copy(pbuf.at[pl.ds(0, 640)], den1.at[pl.ds(nbase, 640)])
    plsc.subcore_barrier()

    for r, (sa_hbm, sb_hbm, ei, wh, den) in enumerate((
            (s00_hbm, s01_hbm, ei0_hbm, wh0_hbm, den0),
            (s10_hbm, s11_hbm, ei1_hbm, wh1_hbm, den1))):
        pltpu.sync_copy(sa_hbm.at[pl.ds(0, N)], s1b.at[pl.ds(0, N)])
        pltpu.sync_copy(sb_hbm.at[pl.ds(0, N)], s2b.at[pl.ds(0, N)])

        pltpu.sync_copy(c_hbm.at[2 * r, pl.ds(0, 16)], cb1)
        pltpu.sync_copy(c_hbm.at[2 * r + 1, pl.ds(0, 16)], cb2)
        msum = cb1[...] + cb2[...]
        c_r = jnp.where(msum >= 0, msum, 0.01 * msum)

        for k in range(KMAX):
            kc = 0 if k < SLOTS else 1
            lbase = (k % SLOTS) * CHUNK
            cid = sub + 16 * k

            @pl.when(cid < NCHUNK)
            def _():
                ebase = cid * CHUNK
                pltpu.async_copy(ei.at[0, pl.ds(ebase, CHUNK)], srcw, esem)
                pltpu.async_copy(ei.at[1, pl.ds(ebase, CHUNK)], dstflat, esem)
                pltpu.make_async_copy(
                    ei.at[0, pl.ds(ebase, CHUNK)], srcw, esem).wait()
                pltpu.make_async_copy(
                    ei.at[1, pl.ds(ebase, CHUNK)], dstflat, esem).wait()

                @plsc.parallel_loop(0, CHUNK, 16, unroll=2)
                def _(t):
                    sv = srcw[pl.ds(t, 16)]
                    dv = dstflat[pl.ds(t, 16)]
                    e = (plsc.load_gather(s1b, [sv])
                         + plsc.load_gather(s2b, [dv]))
                    e = jnp.where(e >= 0, e, 0.01 * e)
                    ptmp[pl.ds(t, 16)] = jnp.exp(e - c_r)

                @pl.when(core == kc)
                def _():
                    @plsc.parallel_loop(0, CHUNK, 16, unroll=4)
                    def _(t):
                        pbuf[pl.ds(lbase + t, 16)] = ptmp[pl.ds(t, 16)]

                pltpu.sync_copy(ptmp, den.at[dstflat], add=True)

        plsc.subcore_barrier()

        pltpu.sync_copy(den.at[pl.ds(nbase, 640)], s1b.at[pl.ds(0, 640)])

        @pl.loop(0, 640, step=16)
        def _(q):
            s1b[pl.ds(q, 16)] = 1.0 / s1b[pl.ds(q, 16)]

        pltpu.sync_copy(s1b.at[pl.ds(0, 640)], den.at[pl.ds(nbase, 640)])
        plsc.subcore_barrier()

        pltpu.sync_copy(den.at[pl.ds(0, N)], s1b.at[pl.ds(0, N)])

        def _scale(buf, base):
            @plsc.parallel_loop(0, BATCH, 2, unroll=2)
            def _(i):
                av0 = plsc.load_gather(
                    pbuf, [jnp.full((16,), base + i, jnp.int32)])
                av1 = plsc.load_gather(
                    pbuf, [jnp.full((16,), base + i + 1, jnp.int32)])
                for q in range(D // 16):
                    buf[i, pl.ds(16 * q, 16)] = (
                        buf[i, pl.ds(16 * q, 16)] * av0)
                for q in range(D // 16):
                    buf[i + 1, pl.ds(16 * q, 16)] = (
                        buf[i + 1, pl.ds(16 * q, 16)] * av1)

        def _wait_gather(buf, sem):
            pltpu.make_async_copy(
                wh.at[srcw.at[pl.ds(0, BATCH)]], buf, sem).wait()

        def _wait_scatter(buf, sem):
            pltpu.make_async_copy(
                buf, numer.at[dstflat.at[pl.ds(0, BATCH)]], sem).wait()

        for k in range(KMAX):
            kc = 0 if k < SLOTS else 1
            lbase = (k % SLOTS) * CHUNK
            cid = sub + 16 * k

            @pl.when(jnp.logical_and(core == kc, cid < NCHUNK))
            def _():
                ebase = cid * CHUNK
                pltpu.async_copy(ei.at[0, pl.ds(ebase, CHUNK)], srcw, esem)
                pltpu.async_copy(ei.at[1, pl.ds(ebase, CHUNK)], dstflat, esem)
                pltpu.make_async_copy(
                    ei.at[0, pl.ds(ebase, CHUNK)], srcw, esem).wait()
                pltpu.make_async_copy(
                    ei.at[1, pl.ds(ebase, CHUNK)], dstflat, esem).wait()

                @plsc.parallel_loop(0, CHUNK, 16, unroll=2)
                def _(t):
                    dv = dstflat[pl.ds(t, 16)]
                    iv = plsc.load_gather(s1b, [dv])
                    pbuf[pl.ds(lbase + t, 16)] = (
                        pbuf[pl.ds(lbase + t, 16)] * iv)

                pltpu.async_copy(wh.at[srcw.at[pl.ds(0, BATCH)]], r_a, gsem_a)

                @pl.loop(0, NB, step=2)
                def _(j):
                    @pl.when(j > 0)
                    def _():
                        _wait_scatter(r_b, ssem_b)

                    pltpu.async_copy(
                        wh.at[srcw.at[pl.ds(BATCH * (j + 1), BATCH)]],
                        r_b, gsem_b)
                    _wait_gather(r_a, gsem_a)
                    _scale(r_a, lbase + BATCH * j)
                    pltpu.async_copy(
                        r_a, numer.at[dstflat.at[pl.ds(BATCH * j, BATCH)]],
                        ssem_a, add=True)

                    @pl.when(j + 2 < NB)
                    def _():
                        _wait_scatter(r_a, ssem_a)
                        pltpu.async_copy(
                            wh.at[srcw.at[pl.ds(BATCH * (j + 2), BATCH)]],
                            r_a, gsem_a)

                    _wait_gather(r_b, gsem_b)
                    _scale(r_b, lbase + BATCH * (j + 1))
                    pltpu.async_copy(
                        r_b,
                        numer.at[dstflat.at[pl.ds(BATCH * (j + 1), BATCH)]],
                        ssem_b, add=True)

                _wait_scatter(r_a, ssem_a)
                _wait_scatter(r_b, ssem_b)

    plsc.subcore_barrier()

    @pl.when(sub < 15)
    def _():
        ob = sub * 624
        pltpu.sync_copy(numer.at[pl.ds(ob, 624), :],
                        out_hbm.at[core, pl.ds(ob, 624), :])

    @pl.when(sub == 15)
    def _():
        pltpu.sync_copy(numer.at[pl.ds(9360, 640), :],
                        out_hbm.at[core, pl.ds(9360, 640), :])


_sc_mesh = plsc.VectorSubcoreMesh(core_axis_name="core",
                                  subcore_axis_name="subcore")

_sc_params = pltpu.CompilerParams()
if "needs_layout_passes" in pltpu.CompilerParams.__dataclass_fields__:
    _sc_params = dataclasses.replace(_sc_params, needs_layout_passes=False)

_sc_layer = pl.kernel(
    _sc_layer_body,
    out_type=jax.ShapeDtypeStruct((2, N, D), jnp.float32),
    mesh=_sc_mesh,
    compiler_params=_sc_params,
    scratch_types=[
        pltpu.VMEM((N,), jnp.float32),
        pltpu.VMEM((N,), jnp.float32),
        pltpu.VMEM((CHUNK,), jnp.int32),
        pltpu.VMEM((CHUNK,), jnp.int32),
        pltpu.VMEM((SLOTS * CHUNK,), jnp.float32),
        pltpu.VMEM((CHUNK,), jnp.float32),
        pltpu.VMEM((16,), jnp.float32),
        pltpu.VMEM((16,), jnp.float32),
        pltpu.VMEM((BATCH, D), jnp.float32),
        pltpu.VMEM((BATCH, D), jnp.float32),
        pltpu.VMEM_SHARED((NPAD, D), jnp.float32),
        pltpu.VMEM_SHARED((NPAD,), jnp.float32),
        pltpu.VMEM_SHARED((NPAD,), jnp.float32),
        pltpu.SemaphoreType.DMA,
        pltpu.SemaphoreType.DMA,
        pltpu.SemaphoreType.DMA,
        pltpu.SemaphoreType.DMA,
        pltpu.SemaphoreType.DMA,
    ],
)


def kernel(x, edge_index_rel0, edge_index_rel1,
           W1_rel0, b1_rel0, a1_rel0, W1_rel1, b1_rel1, a1_rel1,
           W2_rel0, b2_rel0, a2_rel0, W2_rel1, b2_rel1, a2_rel1):
    ei0 = edge_index_rel0.astype(jnp.int32)
    ei1 = edge_index_rel1.astype(jnp.int32)

    def prep(b, a):
        return b.reshape(1, D), a.reshape(2, D)

    b10, a10 = prep(b1_rel0, a1_rel0)
    b11, a11 = prep(b1_rel1, a1_rel1)
    b20, a20 = prep(b2_rel0, a2_rel0)
    b21, a21 = prep(b2_rel1, a2_rel1)

    wh10, wh11, s00, s01, s10, s11, c1 = _tc_dense1(
        x, W1_rel0, b10, a10, W1_rel1, b11, a11)
    part1 = _sc_layer(wh10, wh11, s00.reshape(N), s01.reshape(N),
                      s10.reshape(N), s11.reshape(N), c1, ei0, ei1)
    wh20, wh21, t00, t01, t10, t11, c2 = _tc_dense2(
        part1, W2_rel0, b20, a20, W2_rel1, b21, a21)
    part2 = _sc_layer(wh20, wh21, t00.reshape(N), t01.reshape(N),
                      t10.reshape(N), t11.reshape(N), c2, ei0, ei1)
    return _tc_combine(part2)


# speedup vs baseline: 25.1363x; 1.0119x over previous
import dataclasses
import functools

import jax
import jax.numpy as jnp
from jax import lax
from jax.experimental import pallas as pl
from jax.experimental.pallas import tpu as pltpu
from jax.experimental.pallas import tpu_sc as plsc

N = 10000
D = 128
E = 160000
NPAD = 10240
BR = 2000
CHUNK = 1280
NCHUNK = E // CHUNK
SLOTS = 4
KMAX = 8
BATCH = 64
NB = CHUNK // BATCH


def _dense_compute(xb, w0, b0, a0, w1, b1, a1, wh0_ref, wh1_ref,
                   s00_ref, s01_ref, s10_ref, s11_ref, c_ref):
    wh0 = jnp.dot(xb, w0, preferred_element_type=jnp.float32) + b0
    wh1 = jnp.dot(xb, w1, preferred_element_type=jnp.float32) + b1
    wh0_ref[...] = wh0
    wh1_ref[...] = wh1
    dn = (((1,), (1,)), ((), ()))
    v00 = lax.dot_general(wh0, a0[0:1, :], dn,
                          preferred_element_type=jnp.float32)
    v01 = lax.dot_general(wh0, a0[1:2, :], dn,
                          preferred_element_type=jnp.float32)
    v10 = lax.dot_general(wh1, a1[0:1, :], dn,
                          preferred_element_type=jnp.float32)
    v11 = lax.dot_general(wh1, a1[1:2, :], dn,
                          preferred_element_type=jnp.float32)
    s00_ref[...] = v00
    s01_ref[...] = v01
    s10_ref[...] = v10
    s11_ref[...] = v11
    c_blk = jnp.concatenate(
        [jnp.full((1, D), jnp.max(v)) for v in (v00, v01, v10, v11)]
        + [jnp.zeros((4, D), jnp.float32)], axis=0)
    i = pl.program_id(0)

    @pl.when(i == 0)
    def _():
        c_ref[...] = c_blk

    @pl.when(i > 0)
    def _():
        c_ref[...] = jnp.maximum(c_ref[...], c_blk)


def _tc_dense1_body(x_ref, w0_ref, b0_ref, a0_ref, w1_ref, b1_ref, a1_ref,
                    wh0_ref, wh1_ref, s00_ref, s01_ref, s10_ref, s11_ref,
                    c_ref):
    _dense_compute(x_ref[...], w0_ref[...], b0_ref[...], a0_ref[...],
                   w1_ref[...], b1_ref[...], a1_ref[...],
                   wh0_ref, wh1_ref, s00_ref, s01_ref, s10_ref, s11_ref,
                   c_ref)


def _tc_dense2_body(p_ref, w0_ref, b0_ref, a0_ref, w1_ref, b1_ref, a1_ref,
                    wh0_ref, wh1_ref, s00_ref, s01_ref, s10_ref, s11_ref,
                    c_ref):
    h = p_ref[0] + p_ref[1]
    xb = jnp.where(h >= 0, h, 0.01 * h)
    _dense_compute(xb, w0_ref[...], b0_ref[...], a0_ref[...],
                   w1_ref[...], b1_ref[...], a1_ref[...],
                   wh0_ref, wh1_ref, s00_ref, s01_ref, s10_ref, s11_ref,
                   c_ref)


def _tc_combine_body(p_ref, o_ref):
    o_ref[...] = p_ref[0] + p_ref[1]


_W_SPEC = pl.BlockSpec((D, D), lambda i: (0, 0))
_B_SPEC = pl.BlockSpec((1, D), lambda i: (0, 0))
_A_SPEC = pl.BlockSpec((2, D), lambda i: (0, 0))
_ROW_SPEC = pl.BlockSpec((BR, D), lambda i: (i, 0))
_S_SPEC = pl.BlockSpec((BR, 1), lambda i: (i, 0))
_PART_SPEC = pl.BlockSpec((2, BR, D), lambda i: (0, i, 0))

_C_SPEC = pl.BlockSpec((8, D), lambda i: (0, 0))

_DENSE_OUT = (
    jax.ShapeDtypeStruct((N, D), jnp.float32),
    jax.ShapeDtypeStruct((N, D), jnp.float32),
    jax.ShapeDtypeStruct((N, 1), jnp.float32),
    jax.ShapeDtypeStruct((N, 1), jnp.float32),
    jax.ShapeDtypeStruct((N, 1), jnp.float32),
    jax.ShapeDtypeStruct((N, 1), jnp.float32),
    jax.ShapeDtypeStruct((8, D), jnp.float32),
)

_tc_dense1 = pl.pallas_call(
    _tc_dense1_body,
    grid=(N // BR,),
    in_specs=[_ROW_SPEC, _W_SPEC, _B_SPEC, _A_SPEC, _W_SPEC, _B_SPEC, _A_SPEC],
    out_specs=[_ROW_SPEC, _ROW_SPEC, _S_SPEC, _S_SPEC, _S_SPEC, _S_SPEC,
               _C_SPEC],
    out_shape=_DENSE_OUT,
)

_tc_dense2 = pl.pallas_call(
    _tc_dense2_body,
    grid=(N // BR,),
    in_specs=[_PART_SPEC, _W_SPEC, _B_SPEC, _A_SPEC, _W_SPEC, _B_SPEC, _A_SPEC],
    out_specs=[_ROW_SPEC, _ROW_SPEC, _S_SPEC, _S_SPEC, _S_SPEC, _S_SPEC,
               _C_SPEC],
    out_shape=_DENSE_OUT,
)

_tc_combine = pl.pallas_call(
    _tc_combine_body,
    grid=(N // BR,),
    in_specs=[_PART_SPEC],
    out_specs=_ROW_SPEC,
    out_shape=jax.ShapeDtypeStruct((N, D), jnp.float32),
)


def _sc_layer_body(wh0_hbm, wh1_hbm, s00_hbm, s01_hbm, s10_hbm, s11_hbm,
                   c_hbm, ei0_hbm, ei1_hbm, out_hbm,
                   s1b, s2b, srcw, dstflat, pbuf, ptmp, cb1, cb2, r_a, r_b,
                   numer, den0, den1, esem, gsem_a, gsem_b, ssem_a, ssem_b):
    core = lax.axis_index("core")
    sub = lax.axis_index("subcore")

    zv = jnp.zeros((16,), jnp.float32)

    @pl.loop(0, BATCH)
    def _(i):
        @pl.loop(0, D, step=16)
        def _(q):
            r_a[i, pl.ds(q, 16)] = zv

    @pl.loop(0, 640, step=16)
    def _(q):
        pbuf[pl.ds(q, 16)] = zv

    nbase = sub * 640
    for k in range(640 // BATCH):
        pltpu.sync_copy(r_a.at[:, :],
                        numer.at[pl.ds(nbase + BATCH * k, BATCH), :])
    pltpu.sync_copy(pbuf.at[pl.ds(0, 640)], den0.at[pl.ds(nbase, 640)])
    pltpu.sync_copy(pbuf.at[pl.ds(0, 640)], den1.at[pl.ds(nbase, 640)])
    plsc.subcore_barrier()

    for r, (sa_hbm, sb_hbm, ei, wh, den) in enumerate((
            (s00_hbm, s01_hbm, ei0_hbm, wh0_hbm, den0),
            (s10_hbm, s11_hbm, ei1_hbm, wh1_hbm, den1))):
        pltpu.sync_copy(sa_hbm.at[pl.ds(0, N)], s1b.at[pl.ds(0, N)])
        pltpu.sync_copy(sb_hbm.at[pl.ds(0, N)], s2b.at[pl.ds(0, N)])

        pltpu.sync_copy(c_hbm.at[2 * r, pl.ds(0, 16)], cb1)
        pltpu.sync_copy(c_hbm.at[2 * r + 1, pl.ds(0, 16)], cb2)
        msum = cb1[...] + cb2[...]
        c_r = jnp.where(msum >= 0, msum, 0.01 * msum)

        for k in range(KMAX):
            kc = 0 if k < SLOTS else 1
            lbase = (k % SLOTS) * CHUNK
            cid = sub + 16 * k

            @pl.when(cid < NCHUNK)
            def _():
                ebase = cid * CHUNK
                pltpu.async_copy(ei.at[0, pl.ds(ebase, CHUNK)], srcw, esem)
                pltpu.async_copy(ei.at[1, pl.ds(ebase, CHUNK)], dstflat, esem)
                pltpu.make_async_copy(
                    ei.at[0, pl.ds(ebase, CHUNK)], srcw, esem).wait()
                pltpu.make_async_copy(
                    ei.at[1, pl.ds(ebase, CHUNK)], dstflat, esem).wait()

                @plsc.parallel_loop(0, CHUNK, 16, unroll=4)
                def _(t):
                    sv = srcw[pl.ds(t, 16)]
                    dv = dstflat[pl.ds(t, 16)]
                    e = (plsc.load_gather(s1b, [sv])
                         + plsc.load_gather(s2b, [dv]))
                    e = jnp.where(e >= 0, e, 0.01 * e)
                    ptmp[pl.ds(t, 16)] = jnp.exp(e - c_r)

                @pl.when(core == kc)
                def _():
                    @plsc.parallel_loop(0, CHUNK, 16, unroll=4)
                    def _(t):
                        pbuf[pl.ds(lbase + t, 16)] = ptmp[pl.ds(t, 16)]

                pltpu.sync_copy(ptmp, den.at[dstflat], add=True)

        plsc.subcore_barrier()

        pltpu.sync_copy(den.at[pl.ds(nbase, 640)], s1b.at[pl.ds(0, 640)])

        @pl.loop(0, 640, step=16)
        def _(q):
            s1b[pl.ds(q, 16)] = 1.0 / s1b[pl.ds(q, 16)]

        pltpu.sync_copy(s1b.at[pl.ds(0, 640)], den.at[pl.ds(nbase, 640)])
        plsc.subcore_barrier()

        pltpu.sync_copy(den.at[pl.ds(0, N)], s1b.at[pl.ds(0, N)])

        def _scale(buf, base):
            @plsc.parallel_loop(0, BATCH, 2, unroll=2)
            def _(i):
                av0 = plsc.load_gather(
                    pbuf, [jnp.full((16,), base + i, jnp.int32)])
                av1 = plsc.load_gather(
                    pbuf, [jnp.full((16,), base + i + 1, jnp.int32)])
                for q in range(D // 16):
                    buf[i, pl.ds(16 * q, 16)] = (
                        buf[i, pl.ds(16 * q, 16)] * av0)
                for q in range(D // 16):
                    buf[i + 1, pl.ds(16 * q, 16)] = (
                        buf[i + 1, pl.ds(16 * q, 16)] * av1)

        def _wait_gather(buf, sem):
            pltpu.make_async_copy(
                wh.at[srcw.at[pl.ds(0, BATCH)]], buf, sem).wait()

        def _wait_scatter(buf, sem):
            pltpu.make_async_copy(
                buf, numer.at[dstflat.at[pl.ds(0, BATCH)]], sem).wait()

        for k in range(KMAX):
            kc = 0 if k < SLOTS else 1
            lbase = (k % SLOTS) * CHUNK
            cid = sub + 16 * k

            @pl.when(jnp.logical_and(core == kc, cid < NCHUNK))
            def _():
                ebase = cid * CHUNK
                pltpu.async_copy(ei.at[0, pl.ds(ebase, CHUNK)], srcw, esem)
                pltpu.async_copy(ei.at[1, pl.ds(ebase, CHUNK)], dstflat, esem)
                pltpu.make_async_copy(
                    ei.at[0, pl.ds(ebase, CHUNK)], srcw, esem).wait()
                pltpu.make_async_copy(
                    ei.at[1, pl.ds(ebase, CHUNK)], dstflat, esem).wait()

                @plsc.parallel_loop(0, CHUNK, 16, unroll=2)
                def _(t):
                    dv = dstflat[pl.ds(t, 16)]
                    iv = plsc.load_gather(s1b, [dv])
                    pbuf[pl.ds(lbase + t, 16)] = (
                        pbuf[pl.ds(lbase + t, 16)] * iv)

                pltpu.async_copy(wh.at[srcw.at[pl.ds(0, BATCH)]], r_a, gsem_a)

                @pl.loop(0, NB, step=2)
                def _(j):
                    @pl.when(j > 0)
                    def _():
                        _wait_scatter(r_b, ssem_b)

                    pltpu.async_copy(
                        wh.at[srcw.at[pl.ds(BATCH * (j + 1), BATCH)]],
                        r_b, gsem_b)
                    _wait_gather(r_a, gsem_a)
                    _scale(r_a, lbase + BATCH * j)
                    pltpu.async_copy(
                        r_a, numer.at[dstflat.at[pl.ds(BATCH * j, BATCH)]],
                        ssem_a, add=True)

                    @pl.when(j + 2 < NB)
                    def _():
                        _wait_scatter(r_a, ssem_a)
                        pltpu.async_copy(
                            wh.at[srcw.at[pl.ds(BATCH * (j + 2), BATCH)]],
                            r_a, gsem_a)

                    _wait_gather(r_b, gsem_b)
                    _scale(r_b, lbase + BATCH * (j + 1))
                    pltpu.async_copy(
                        r_b,
                        numer.at[dstflat.at[pl.ds(BATCH * (j + 1), BATCH)]],
                        ssem_b, add=True)

                _wait_scatter(r_a, ssem_a)
                _wait_scatter(r_b, ssem_b)

    plsc.subcore_barrier()

    @pl.when(sub < 15)
    def _():
        ob = sub * 624
        pltpu.sync_copy(numer.at[pl.ds(ob, 624), :],
                        out_hbm.at[core, pl.ds(ob, 624), :])

    @pl.when(sub == 15)
    def _():
        pltpu.sync_copy(numer.at[pl.ds(9360, 640), :],
                        out_hbm.at[core, pl.ds(9360, 640), :])


_sc_mesh = plsc.VectorSubcoreMesh(core_axis_name="core",
                                  subcore_axis_name="subcore")

_sc_params = pltpu.CompilerParams()
if "needs_layout_passes" in pltpu.CompilerParams.__dataclass_fields__:
    _sc_params = dataclasses.replace(_sc_params, needs_layout_passes=False)

_sc_layer = pl.kernel(
    _sc_layer_body,
    out_type=jax.ShapeDtypeStruct((2, N, D), jnp.float32),
    mesh=_sc_mesh,
    compiler_params=_sc_params,
    scratch_types=[
        pltpu.VMEM((N,), jnp.float32),
        pltpu.VMEM((N,), jnp.float32),
        pltpu.VMEM((CHUNK,), jnp.int32),
        pltpu.VMEM((CHUNK,), jnp.int32),
        pltpu.VMEM((SLOTS * CHUNK,), jnp.float32),
        pltpu.VMEM((CHUNK,), jnp.float32),
        pltpu.VMEM((16,), jnp.float32),
        pltpu.VMEM((16,), jnp.float32),
        pltpu.VMEM((BATCH, D), jnp.float32),
        pltpu.VMEM((BATCH, D), jnp.float32),
        pltpu.VMEM_SHARED((NPAD, D), jnp.float32),
        pltpu.VMEM_SHARED((NPAD,), jnp.float32),
        pltpu.VMEM_SHARED((NPAD,), jnp.float32),
        pltpu.SemaphoreType.DMA,
        pltpu.SemaphoreType.DMA,
        pltpu.SemaphoreType.DMA,
        pltpu.SemaphoreType.DMA,
        pltpu.SemaphoreType.DMA,
    ],
)


def kernel(x, edge_index_rel0, edge_index_rel1,
           W1_rel0, b1_rel0, a1_rel0, W1_rel1, b1_rel1, a1_rel1,
           W2_rel0, b2_rel0, a2_rel0, W2_rel1, b2_rel1, a2_rel1):
    ei0 = edge_index_rel0.astype(jnp.int32)
    ei1 = edge_index_rel1.astype(jnp.int32)

    def prep(b, a):
        return b.reshape(1, D), a.reshape(2, D)

    b10, a10 = prep(b1_rel0, a1_rel0)
    b11, a11 = prep(b1_rel1, a1_rel1)
    b20, a20 = prep(b2_rel0, a2_rel0)
    b21, a21 = prep(b2_rel1, a2_rel1)

    wh10, wh11, s00, s01, s10, s11, c1 = _tc_dense1(
        x, W1_rel0, b10, a10, W1_rel1, b11, a11)
    part1 = _sc_layer(wh10, wh11, s00.reshape(N), s01.reshape(N),
                      s10.reshape(N), s11.reshape(N), c1, ei0, ei1)
    wh20, wh21, t00, t01, t10, t11, c2 = _tc_dense2(
        part1, W2_rel0, b20, a20, W2_rel1, b21, a21)
    part2 = _sc_layer(wh20, wh21, t00.reshape(N), t01.reshape(N),
                      t10.reshape(N), t11.reshape(N), c2, ei0, ei1)
    return _tc_combine(part2)
